# baseline (speedup 1.0000x reference)
import functools

import jax
import jax.numpy as jnp
from jax import lax
from jax.experimental import pallas as pl
from jax.experimental.pallas import tpu as pltpu


def _wass_body(x_ref, y_ref, xw_ref, yw_ref, out_ref, vbuf, wbuf, *,
               bs):
    RT = vbuf.shape[0]
    C = vbuf.shape[1]
    HRT = RT // 2
    rb = RT.bit_length() - 1
    logm = (RT * C).bit_length() - 1
    bslog = bs.bit_length() - 1
    nb = RT // bs
    nstages = logm * (logm + 1) // 2
    f32 = jnp.float32
    i32 = jnp.int32

    def fill_tot(b, acc):
        ax, ay = acc
        ax = ax + jnp.sum(xw_ref[pl.ds(b * bs, bs), :])
        ay = ay + jnp.sum(yw_ref[pl.ds(b * bs, bs), :])
        vbuf[pl.ds(b * bs, bs), :] = x_ref[pl.ds(b * bs, bs), :]
        vbuf[pl.ds(HRT + b * bs, bs), :] = y_ref[pl.ds(b * bs, bs), :]
        return ax, ay

    wx_tot, wy_tot = lax.fori_loop(0, nb // 2, fill_tot,
                                   (jnp.float32(0.0), jnp.float32(0.0)))
    inv_x = 1.0 / wx_tot
    inv_y = -1.0 / wy_tot

    def fill_w(b, _):
        wbuf[pl.ds(b * bs, bs), :] = xw_ref[pl.ds(b * bs, bs), :] * inv_x
        wbuf[pl.ds(HRT + b * bs, bs), :] = yw_ref[pl.ds(b * bs, bs), :] * inv_y
        return 0

    lax.fori_loop(0, nb // 2, fill_w, 0)

    rl_iota = lax.broadcasted_iota(i32, (bs, C), 0)
    c_iota = lax.broadcasted_iota(i32, (bs, C), 1)

    def desc_mask(grow, s):
        return jnp.where(
            s < rb,
            (grow >> jnp.minimum(s, rb - 1)) & 1,
            (c_iota >> jnp.maximum(s - rb, 0)) & 1,
        )

    def roll_substage(axis, size, jdist, hb, db, base):
        vv = vbuf[pl.ds(base, bs), :]
        ww = wbuf[pl.ds(base, bs), :]
        hbf = hb.astype(f32)
        sgn = ((1 - 2 * hb) * (1 - 2 * db)).astype(f32)
        pv = hbf * pltpu.roll(vv, jdist, axis) + \
            (1.0 - hbf) * pltpu.roll(vv, size - jdist, axis)
        pw = hbf * pltpu.roll(ww, jdist, axis) + \
            (1.0 - hbf) * pltpu.roll(ww, size - jdist, axis)
        ex = sgn * (vv - pv) > 0
        vbuf[pl.ds(base, bs), :] = jnp.where(ex, pv, vv)
        wbuf[pl.ds(base, bs), :] = jnp.where(ex, pw, ww)

    def stage_body(k, _):
        kf = (1 + 8 * k).astype(f32)
        s = jnp.floor((1.0 + jnp.sqrt(kf)) * 0.5).astype(i32)
        t = s - 1 - (k - (s * (s - 1)) // 2)

        @pl.when(t < bslog)
        def _row_roll():
            def blk(b, _):
                base = b * bs
                j = 1 << t
                hb = (rl_iota >> t) & 1
                db = desc_mask(rl_iota + base, s)
                roll_substage(0, bs, j, hb, db, base)
                return 0
            lax.fori_loop(0, nb, blk, 0)

        @pl.when(jnp.logical_and(t >= bslog, t < rb))
        def _block_pair():
            jb = 1 << (t - bslog)

            def blk(p, _):
                @pl.when((p & jb) == 0)
                def _():
                    q = p | jb
                    av = vbuf[pl.ds(p * bs, bs), :]
                    aw = wbuf[pl.ds(p * bs, bs), :]
                    bv = vbuf[pl.ds(q * bs, bs), :]
                    bw = wbuf[pl.ds(q * bs, bs), :]
                    db = desc_mask(rl_iota + p * bs, s)
                    sgn = (1 - 2 * db).astype(f32)
                    ex = sgn * (av - bv) > 0
                    vbuf[pl.ds(p * bs, bs), :] = jnp.where(ex, bv, av)
                    wbuf[pl.ds(p * bs, bs), :] = jnp.where(ex, bw, aw)
                    vbuf[pl.ds(q * bs, bs), :] = jnp.where(ex, av, bv)
                    wbuf[pl.ds(q * bs, bs), :] = jnp.where(ex, aw, bw)
                return 0

            lax.fori_loop(0, nb, blk, 0)

        @pl.when(t >= rb)
        def _lane_roll():
            jl = 1 << (t - rb)

            def blk(b, _):
                hb = (c_iota >> (t - rb)) & 1
                db = (c_iota >> jnp.maximum(s - rb, 0)) & 1
                roll_substage(1, C, jl, hb, db, b * bs)
                return 0

            lax.fori_loop(0, nb, blk, 0)

        return 0

    lax.fori_loop(0, nstages, stage_body, 0)

    def local_scan(ww):
        n = 1
        while n < bs:
            ww = ww + jnp.concatenate(
                [jnp.zeros((n, C), f32), ww[: bs - n, :]], axis=0)
            n *= 2
        return ww

    def pass1(b, carry):
        ww = wbuf[pl.ds(b * bs, bs), :]
        incl = local_scan(ww)
        return carry + incl[bs - 1: bs, :]

    col_tot = lax.fori_loop(0, nb, pass1, jnp.zeros((1, C), f32))

    lane_incl = col_tot
    n = 1
    while n < C:
        lane_incl = lane_incl + jnp.concatenate(
            [jnp.zeros((1, n), f32), lane_incl[:, : C - n]], axis=1)
        n *= 2
    col_off = lane_incl - col_tot

    def pass2(b, carry):
        off, loss = carry
        ww = wbuf[pl.ds(b * bs, bs), :]
        vv = vbuf[pl.ds(b * bs, bs), :]
        incl_local = local_scan(ww)
        s_incl = incl_local + off
        s_excl = s_incl - ww
        loss = loss + jnp.sum((jnp.abs(s_excl) - jnp.abs(s_incl)) * vv)
        return off + incl_local[bs - 1: bs, :], loss

    _, loss = lax.fori_loop(
        0, nb, pass2, (col_off, jnp.float32(0.0)))
    out_ref[...] = loss.reshape(1, 1)


@functools.partial(jax.jit, static_argnames=("rows", "cols", "bs", "interpret"))
def _wass_loss(x, y, xw, yw, rows, cols, bs=1024, interpret=False):
    body = functools.partial(_wass_body, bs=bs)
    f = pl.pallas_call(
        body,
        out_shape=jax.ShapeDtypeStruct((1, 1), jnp.float32),
        scratch_shapes=[
            pltpu.VMEM((2 * rows, cols), jnp.float32),
            pltpu.VMEM((2 * rows, cols), jnp.float32),
        ],
        interpret=interpret,
    )
    return f(
        x.reshape(rows, cols),
        y.reshape(rows, cols),
        xw.reshape(rows, cols),
        yw.reshape(rows, cols),
    )[0, 0]


def kernel(x, y, x_weights, y_weights, pre_sorted):
    del pre_sorted
    n = x.shape[0]
    cols = 128
    rows = n // cols
    return _wass_loss(x, y, x_weights, y_weights, rows, cols, bs=2048)


# speedup vs baseline: 1.6175x; 1.6175x over previous
import functools

import jax
import jax.numpy as jnp
from jax import lax
from jax.experimental import pallas as pl
from jax.experimental.pallas import tpu as pltpu


def _wass_body(x_ref, y_ref, xw_ref, yw_ref, out_ref, vbuf, wbuf, *,
               bs):
    RT = vbuf.shape[0]
    C = vbuf.shape[1]
    HRT = RT // 2
    rb = RT.bit_length() - 1
    logm = (RT * C).bit_length() - 1
    bslog = bs.bit_length() - 1
    nb = RT // bs
    nstages = logm * (logm + 1) // 2
    f32 = jnp.float32
    i32 = jnp.int32

    def fill_tot(b, acc):
        ax, ay = acc
        ax = ax + jnp.sum(xw_ref[pl.ds(b * bs, bs), :])
        ay = ay + jnp.sum(yw_ref[pl.ds(b * bs, bs), :])
        vbuf[pl.ds(b * bs, bs), :] = x_ref[pl.ds(b * bs, bs), :]
        vbuf[pl.ds(HRT + b * bs, bs), :] = y_ref[pl.ds(b * bs, bs), :]
        return ax, ay

    wx_tot, wy_tot = lax.fori_loop(0, nb // 2, fill_tot,
                                   (jnp.float32(0.0), jnp.float32(0.0)))
    inv_x = 1.0 / wx_tot
    inv_y = -1.0 / wy_tot

    def fill_w(b, _):
        wbuf[pl.ds(b * bs, bs), :] = xw_ref[pl.ds(b * bs, bs), :] * inv_x
        wbuf[pl.ds(HRT + b * bs, bs), :] = yw_ref[pl.ds(b * bs, bs), :] * inv_y
        return 0

    lax.fori_loop(0, nb // 2, fill_w, 0)

    rl_iota = lax.broadcasted_iota(i32, (bs, C), 0)
    c_iota = lax.broadcasted_iota(i32, (bs, C), 1)

    def desc_mask(grow, s):
        return jnp.where(
            s < rb,
            (grow >> jnp.minimum(s, rb - 1)) & 1,
            (c_iota >> jnp.maximum(s - rb, 0)) & 1,
        )

    def roll_substage(axis, size, jdist, hb, db, base):
        vv = vbuf[pl.ds(base, bs), :]
        ww = wbuf[pl.ds(base, bs), :]
        hbf = hb.astype(f32)
        sgn = ((1 - 2 * hb) * (1 - 2 * db)).astype(f32)
        pv = hbf * pltpu.roll(vv, jdist, axis) + \
            (1.0 - hbf) * pltpu.roll(vv, size - jdist, axis)
        pw = hbf * pltpu.roll(ww, jdist, axis) + \
            (1.0 - hbf) * pltpu.roll(ww, size - jdist, axis)
        ex = sgn * (vv - pv) > 0
        vbuf[pl.ds(base, bs), :] = jnp.where(ex, pv, vv)
        wbuf[pl.ds(base, bs), :] = jnp.where(ex, pw, ww)

    def stage_body(k, _):
        kf = (1 + 8 * k).astype(f32)
        s = jnp.floor((1.0 + jnp.sqrt(kf)) * 0.5).astype(i32)
        t = s - 1 - (k - (s * (s - 1)) // 2)

        @pl.when(t < bslog)
        def _row_roll():
            def blk(b, _):
                base = b * bs
                j = 1 << t
                hb = (rl_iota >> t) & 1
                db = desc_mask(rl_iota + base, s)
                roll_substage(0, bs, j, hb, db, base)
                return 0
            lax.fori_loop(0, nb, blk, 0)

        @pl.when(jnp.logical_and(t >= bslog, t < rb))
        def _block_pair():
            jb = 1 << (t - bslog)

            def blk(p, _):
                @pl.when((p & jb) == 0)
                def _():
                    q = p | jb
                    av = vbuf[pl.ds(p * bs, bs), :]
                    aw = wbuf[pl.ds(p * bs, bs), :]
                    bv = vbuf[pl.ds(q * bs, bs), :]
                    bw = wbuf[pl.ds(q * bs, bs), :]
                    db = desc_mask(rl_iota + p * bs, s)
                    sgn = (1 - 2 * db).astype(f32)
                    ex = sgn * (av - bv) > 0
                    vbuf[pl.ds(p * bs, bs), :] = jnp.where(ex, bv, av)
                    wbuf[pl.ds(p * bs, bs), :] = jnp.where(ex, bw, aw)
                    vbuf[pl.ds(q * bs, bs), :] = jnp.where(ex, av, bv)
                    wbuf[pl.ds(q * bs, bs), :] = jnp.where(ex, aw, bw)
                return 0

            lax.fori_loop(0, nb, blk, 0)

        @pl.when(t >= rb)
        def _lane_roll():
            jl = 1 << (t - rb)

            def blk(b, _):
                hb = (c_iota >> (t - rb)) & 1
                db = (c_iota >> jnp.maximum(s - rb, 0)) & 1
                roll_substage(1, C, jl, hb, db, b * bs)
                return 0

            lax.fori_loop(0, nb, blk, 0)

        return 0

    lax.fori_loop(0, nstages, stage_body, 0)

    def local_scan(ww):
        n = 1
        while n < bs:
            ww = ww + jnp.concatenate(
                [jnp.zeros((n, C), f32), ww[: bs - n, :]], axis=0)
            n *= 2
        return ww

    def pass1(b, carry):
        ww = wbuf[pl.ds(b * bs, bs), :]
        incl = local_scan(ww)
        return carry + incl[bs - 1: bs, :]

    col_tot = lax.fori_loop(0, nb, pass1, jnp.zeros((1, C), f32))

    lane_incl = col_tot
    n = 1
    while n < C:
        lane_incl = lane_incl + jnp.concatenate(
            [jnp.zeros((1, n), f32), lane_incl[:, : C - n]], axis=1)
        n *= 2
    col_off = lane_incl - col_tot

    def pass2(b, carry):
        off, loss = carry
        ww = wbuf[pl.ds(b * bs, bs), :]
        vv = vbuf[pl.ds(b * bs, bs), :]
        incl_local = local_scan(ww)
        s_incl = incl_local + off
        s_excl = s_incl - ww
        loss = loss + jnp.sum((jnp.abs(s_excl) - jnp.abs(s_incl)) * vv)
        return off + incl_local[bs - 1: bs, :], loss

    _, loss = lax.fori_loop(
        0, nb, pass2, (col_off, jnp.float32(0.0)))
    out_ref[...] = loss.reshape(1, 1)


@functools.partial(jax.jit, static_argnames=("rows", "cols", "bs", "interpret"))
def _wass_loss(x, y, xw, yw, rows, cols, bs=1024, interpret=False):
    body = functools.partial(_wass_body, bs=bs)
    f = pl.pallas_call(
        body,
        out_shape=jax.ShapeDtypeStruct((1, 1), jnp.float32),
        scratch_shapes=[
            pltpu.VMEM((2 * rows, cols), jnp.float32),
            pltpu.VMEM((2 * rows, cols), jnp.float32),
        ],
        interpret=interpret,
    )
    return f(
        x.reshape(rows, cols),
        y.reshape(rows, cols),
        xw.reshape(rows, cols),
        yw.reshape(rows, cols),
    )[0, 0]


def kernel(x, y, x_weights, y_weights, pre_sorted):
    del pre_sorted
    n = x.shape[0]
    cols = 128
    rows = n // cols
    return _wass_loss(x, y, x_weights, y_weights, rows, cols, bs=512)


# speedup vs baseline: 2.0260x; 1.2526x over previous
import functools

import jax
import jax.numpy as jnp
from jax import lax
from jax.experimental import pallas as pl
from jax.experimental.pallas import tpu as pltpu


def _wass_body(x_ref, y_ref, xw_ref, yw_ref, out_ref, vbuf, wbuf, *,
               bs):
    RT = vbuf.shape[0]
    C = vbuf.shape[1]
    HRT = RT // 2
    rb = RT.bit_length() - 1
    logm = (RT * C).bit_length() - 1
    bslog = bs.bit_length() - 1
    nb = RT // bs
    nstages = logm * (logm + 1) // 2
    f32 = jnp.float32
    i32 = jnp.int32

    def fill_tot(b, acc):
        ax, ay = acc
        ax = ax + jnp.sum(xw_ref[pl.ds(b * bs, bs), :])
        ay = ay + jnp.sum(yw_ref[pl.ds(b * bs, bs), :])
        vbuf[pl.ds(b * bs, bs), :] = x_ref[pl.ds(b * bs, bs), :]
        vbuf[pl.ds(HRT + b * bs, bs), :] = y_ref[pl.ds(b * bs, bs), :]
        return ax, ay

    wx_tot, wy_tot = lax.fori_loop(0, nb // 2, fill_tot,
                                   (jnp.float32(0.0), jnp.float32(0.0)))
    inv_x = 1.0 / wx_tot
    inv_y = -1.0 / wy_tot

    def fill_w(b, _):
        wbuf[pl.ds(b * bs, bs), :] = xw_ref[pl.ds(b * bs, bs), :] * inv_x
        wbuf[pl.ds(HRT + b * bs, bs), :] = yw_ref[pl.ds(b * bs, bs), :] * inv_y
        return 0

    lax.fori_loop(0, nb // 2, fill_w, 0)

    rl_iota = lax.broadcasted_iota(i32, (bs, C), 0)
    c_iota = lax.broadcasted_iota(i32, (bs, C), 1)

    def desc_mask(grow, s):
        return jnp.where(
            s < rb,
            (grow >> jnp.minimum(s, rb - 1)) & 1,
            (c_iota >> jnp.maximum(s - rb, 0)) & 1,
        )

    def roll_substage(axis, size, jdist, hb, db, base):
        vv = vbuf[pl.ds(base, bs), :]
        ww = wbuf[pl.ds(base, bs), :]
        hbf = hb.astype(f32)
        sgn = ((1 - 2 * hb) * (1 - 2 * db)).astype(f32)
        pv = hbf * pltpu.roll(vv, jdist, axis) + \
            (1.0 - hbf) * pltpu.roll(vv, size - jdist, axis)
        pw = hbf * pltpu.roll(ww, jdist, axis) + \
            (1.0 - hbf) * pltpu.roll(ww, size - jdist, axis)
        ex = sgn * (vv - pv) > 0
        vbuf[pl.ds(base, bs), :] = jnp.where(ex, pv, vv)
        wbuf[pl.ds(base, bs), :] = jnp.where(ex, pw, ww)

    def stage_body(k, _):
        kf = (1 + 8 * k).astype(f32)
        s = jnp.floor((1.0 + jnp.sqrt(kf)) * 0.5).astype(i32)
        t = s - 1 - (k - (s * (s - 1)) // 2)

        @pl.when(t < bslog)
        def _row_roll():
            def blk(b, _):
                base = b * bs
                j = 1 << t
                hb = (rl_iota >> t) & 1
                db = desc_mask(rl_iota + base, s)
                roll_substage(0, bs, j, hb, db, base)
                return 0
            lax.fori_loop(0, nb, blk, 0)

        @pl.when(jnp.logical_and(t >= bslog, t < rb))
        def _block_pair():
            jb = 1 << (t - bslog)

            def blk(p, _):
                @pl.when((p & jb) == 0)
                def _():
                    q = p | jb
                    av = vbuf[pl.ds(p * bs, bs), :]
                    aw = wbuf[pl.ds(p * bs, bs), :]
                    bv = vbuf[pl.ds(q * bs, bs), :]
                    bw = wbuf[pl.ds(q * bs, bs), :]
                    db = desc_mask(rl_iota + p * bs, s)
                    sgn = (1 - 2 * db).astype(f32)
                    ex = sgn * (av - bv) > 0
                    vbuf[pl.ds(p * bs, bs), :] = jnp.where(ex, bv, av)
                    wbuf[pl.ds(p * bs, bs), :] = jnp.where(ex, bw, aw)
                    vbuf[pl.ds(q * bs, bs), :] = jnp.where(ex, av, bv)
                    wbuf[pl.ds(q * bs, bs), :] = jnp.where(ex, aw, bw)
                return 0

            lax.fori_loop(0, nb, blk, 0)

        @pl.when(t >= rb)
        def _lane_roll():
            jl = 1 << (t - rb)

            def blk(b, _):
                hb = (c_iota >> (t - rb)) & 1
                db = (c_iota >> jnp.maximum(s - rb, 0)) & 1
                roll_substage(1, C, jl, hb, db, b * bs)
                return 0

            lax.fori_loop(0, nb, blk, 0)

        return 0

    lax.fori_loop(0, nstages, stage_body, 0)

    def local_scan(ww):
        n = 1
        while n < bs:
            ww = ww + jnp.concatenate(
                [jnp.zeros((n, C), f32), ww[: bs - n, :]], axis=0)
            n *= 2
        return ww

    def pass1(b, carry):
        ww = wbuf[pl.ds(b * bs, bs), :]
        incl = local_scan(ww)
        return carry + incl[bs - 1: bs, :]

    col_tot = lax.fori_loop(0, nb, pass1, jnp.zeros((1, C), f32))

    lane_incl = col_tot
    n = 1
    while n < C:
        lane_incl = lane_incl + jnp.concatenate(
            [jnp.zeros((1, n), f32), lane_incl[:, : C - n]], axis=1)
        n *= 2
    col_off = lane_incl - col_tot

    def pass2(b, carry):
        off, loss = carry
        ww = wbuf[pl.ds(b * bs, bs), :]
        vv = vbuf[pl.ds(b * bs, bs), :]
        incl_local = local_scan(ww)
        s_incl = incl_local + off
        s_excl = s_incl - ww
        loss = loss + jnp.sum((jnp.abs(s_excl) - jnp.abs(s_incl)) * vv)
        return off + incl_local[bs - 1: bs, :], loss

    _, loss = lax.fori_loop(
        0, nb, pass2, (col_off, jnp.float32(0.0)))
    out_ref[...] = loss.reshape(1, 1)


@functools.partial(jax.jit, static_argnames=("rows", "cols", "bs", "interpret"))
def _wass_loss(x, y, xw, yw, rows, cols, bs=1024, interpret=False):
    body = functools.partial(_wass_body, bs=bs)
    f = pl.pallas_call(
        body,
        out_shape=jax.ShapeDtypeStruct((1, 1), jnp.float32),
        scratch_shapes=[
            pltpu.VMEM((2 * rows, cols), jnp.float32),
            pltpu.VMEM((2 * rows, cols), jnp.float32),
        ],
        interpret=interpret,
    )
    return f(
        x.reshape(rows, cols),
        y.reshape(rows, cols),
        xw.reshape(rows, cols),
        yw.reshape(rows, cols),
    )[0, 0]


def kernel(x, y, x_weights, y_weights, pre_sorted):
    del pre_sorted
    n = x.shape[0]
    cols = 128
    rows = n // cols
    return _wass_loss(x, y, x_weights, y_weights, rows, cols, bs=256)


# speedup vs baseline: 2.1696x; 1.0709x over previous
import functools

import jax
import jax.numpy as jnp
from jax import lax
from jax.experimental import pallas as pl
from jax.experimental.pallas import tpu as pltpu


def _wass_body(x_ref, y_ref, xw_ref, yw_ref, out_ref, vbuf, wbuf, *,
               bs):
    RT = vbuf.shape[0]
    C = vbuf.shape[1]
    HRT = RT // 2
    rb = RT.bit_length() - 1
    logm = (RT * C).bit_length() - 1
    bslog = bs.bit_length() - 1
    nb = RT // bs
    nstages = logm * (logm + 1) // 2
    f32 = jnp.float32
    i32 = jnp.int32

    def fill_tot(b, acc):
        ax, ay = acc
        ax = ax + jnp.sum(xw_ref[pl.ds(b * bs, bs), :])
        ay = ay + jnp.sum(yw_ref[pl.ds(b * bs, bs), :])
        vbuf[pl.ds(b * bs, bs), :] = x_ref[pl.ds(b * bs, bs), :]
        vbuf[pl.ds(HRT + b * bs, bs), :] = y_ref[pl.ds(b * bs, bs), :]
        return ax, ay

    wx_tot, wy_tot = lax.fori_loop(0, nb // 2, fill_tot,
                                   (jnp.float32(0.0), jnp.float32(0.0)))
    inv_x = 1.0 / wx_tot
    inv_y = -1.0 / wy_tot

    def fill_w(b, _):
        wbuf[pl.ds(b * bs, bs), :] = xw_ref[pl.ds(b * bs, bs), :] * inv_x
        wbuf[pl.ds(HRT + b * bs, bs), :] = yw_ref[pl.ds(b * bs, bs), :] * inv_y
        return 0

    lax.fori_loop(0, nb // 2, fill_w, 0)

    rl_iota = lax.broadcasted_iota(i32, (bs, C), 0)
    c_iota = lax.broadcasted_iota(i32, (bs, C), 1)

    def desc_mask(grow, s):
        return jnp.where(
            s < rb,
            (grow >> jnp.minimum(s, rb - 1)) & 1,
            (c_iota >> jnp.maximum(s - rb, 0)) & 1,
        )

    def roll_substage(axis, size, jdist, hb, db, base):
        vv = vbuf[pl.ds(base, bs), :]
        ww = wbuf[pl.ds(base, bs), :]
        hbf = hb.astype(f32)
        sgn = ((1 - 2 * hb) * (1 - 2 * db)).astype(f32)
        pv = hbf * pltpu.roll(vv, jdist, axis) + \
            (1.0 - hbf) * pltpu.roll(vv, size - jdist, axis)
        pw = hbf * pltpu.roll(ww, jdist, axis) + \
            (1.0 - hbf) * pltpu.roll(ww, size - jdist, axis)
        ex = sgn * (vv - pv) > 0
        vbuf[pl.ds(base, bs), :] = jnp.where(ex, pv, vv)
        wbuf[pl.ds(base, bs), :] = jnp.where(ex, pw, ww)

    def stage_body(k, _):
        kf = (1 + 8 * k).astype(f32)
        s = jnp.floor((1.0 + jnp.sqrt(kf)) * 0.5).astype(i32)
        t = s - 1 - (k - (s * (s - 1)) // 2)

        @pl.when(t < bslog)
        def _row_roll():
            def blk(b, _):
                base = b * bs
                j = 1 << t
                hb = (rl_iota >> t) & 1
                db = desc_mask(rl_iota + base, s)
                roll_substage(0, bs, j, hb, db, base)
                return 0
            lax.fori_loop(0, nb, blk, 0)

        @pl.when(jnp.logical_and(t >= bslog, t < rb))
        def _block_pair():
            jb = 1 << (t - bslog)

            def blk(p, _):
                @pl.when((p & jb) == 0)
                def _():
                    q = p | jb
                    av = vbuf[pl.ds(p * bs, bs), :]
                    aw = wbuf[pl.ds(p * bs, bs), :]
                    bv = vbuf[pl.ds(q * bs, bs), :]
                    bw = wbuf[pl.ds(q * bs, bs), :]
                    db = desc_mask(rl_iota + p * bs, s)
                    sgn = (1 - 2 * db).astype(f32)
                    ex = sgn * (av - bv) > 0
                    vbuf[pl.ds(p * bs, bs), :] = jnp.where(ex, bv, av)
                    wbuf[pl.ds(p * bs, bs), :] = jnp.where(ex, bw, aw)
                    vbuf[pl.ds(q * bs, bs), :] = jnp.where(ex, av, bv)
                    wbuf[pl.ds(q * bs, bs), :] = jnp.where(ex, aw, bw)
                return 0

            lax.fori_loop(0, nb, blk, 0)

        @pl.when(t >= rb)
        def _lane_roll():
            jl = 1 << (t - rb)

            def blk(b, _):
                hb = (c_iota >> (t - rb)) & 1
                db = (c_iota >> jnp.maximum(s - rb, 0)) & 1
                roll_substage(1, C, jl, hb, db, b * bs)
                return 0

            lax.fori_loop(0, nb, blk, 0)

        return 0

    lax.fori_loop(0, nstages, stage_body, 0)

    def local_scan(ww):
        n = 1
        while n < bs:
            ww = ww + jnp.concatenate(
                [jnp.zeros((n, C), f32), ww[: bs - n, :]], axis=0)
            n *= 2
        return ww

    def pass1(b, carry):
        ww = wbuf[pl.ds(b * bs, bs), :]
        incl = local_scan(ww)
        return carry + incl[bs - 1: bs, :]

    col_tot = lax.fori_loop(0, nb, pass1, jnp.zeros((1, C), f32))

    lane_incl = col_tot
    n = 1
    while n < C:
        lane_incl = lane_incl + jnp.concatenate(
            [jnp.zeros((1, n), f32), lane_incl[:, : C - n]], axis=1)
        n *= 2
    col_off = lane_incl - col_tot

    def pass2(b, carry):
        off, loss = carry
        ww = wbuf[pl.ds(b * bs, bs), :]
        vv = vbuf[pl.ds(b * bs, bs), :]
        incl_local = local_scan(ww)
        s_incl = incl_local + off
        s_excl = s_incl - ww
        loss = loss + jnp.sum((jnp.abs(s_excl) - jnp.abs(s_incl)) * vv)
        return off + incl_local[bs - 1: bs, :], loss

    _, loss = lax.fori_loop(
        0, nb, pass2, (col_off, jnp.float32(0.0)))
    out_ref[...] = loss.reshape(1, 1)


@functools.partial(jax.jit, static_argnames=("rows", "cols", "bs", "interpret"))
def _wass_loss(x, y, xw, yw, rows, cols, bs=1024, interpret=False):
    body = functools.partial(_wass_body, bs=bs)
    f = pl.pallas_call(
        body,
        out_shape=jax.ShapeDtypeStruct((1, 1), jnp.float32),
        scratch_shapes=[
            pltpu.VMEM((2 * rows, cols), jnp.float32),
            pltpu.VMEM((2 * rows, cols), jnp.float32),
        ],
        interpret=interpret,
    )
    return f(
        x.reshape(rows, cols),
        y.reshape(rows, cols),
        xw.reshape(rows, cols),
        yw.reshape(rows, cols),
    )[0, 0]


def kernel(x, y, x_weights, y_weights, pre_sorted):
    del pre_sorted
    n = x.shape[0]
    cols = 128
    rows = n // cols
    return _wass_loss(x, y, x_weights, y_weights, rows, cols, bs=128)
